# baseline (speedup 1.0000x reference)
import functools

import jax
import jax.numpy as jnp
from jax import lax
from jax.experimental import pallas as pl
from jax.experimental.pallas import tpu as pltpu
from jax.experimental.pallas import tpu_sc as plsc

N = 10000
E = 320000
F = 128
K = 64

NC = 2
NS = 16
NW = NC * NS
NSLICES = 1
ES = E // NSLICES
EPW = ES // NW
B = 80
CHUNKS = EPW // B
NP = 10240
RPT = NP // NS
BE = 4000


def _dot_t(a, w):
    return lax.dot_general(a, w, (((1,), (1,)), ((), ())),
                           preferred_element_type=jnp.float32)


def _g_body(rbf_ref, wk_ref, out_ref):
    out_ref[...] = _dot_t(rbf_ref[...], wk_ref[...])


def _edge_matmul(rbf, Wk, sl):
    base = sl * (ES // BE)
    return pl.pallas_call(
        _g_body,
        grid=(ES // BE,),
        in_specs=[
            pl.BlockSpec((BE, K), lambda i: (base + i, 0)),
            pl.BlockSpec((F, K), lambda i: (0, 0)),
        ],
        out_specs=pl.BlockSpec((BE, F), lambda i: (i, 0)),
        out_shape=jax.ShapeDtypeStruct((ES, F), jnp.float32),
    )(rbf, Wk)


def _node_body(x_ref, wi_ref, bi_ref, wj_ref, bj_ref, xi_ref, hj_ref):
    xv = x_ref[...]
    xi_ref[...] = _dot_t(xv, wi_ref[...]) + bi_ref[...]
    hj_ref[:N, :] = _dot_t(xv, wj_ref[...]) + bj_ref[...]
    hj_ref[N:, :] = jnp.zeros((NP - N, F), jnp.float32)


def _node_matmuls(x, Wi, bi, Wj, bj):
    return pl.pallas_call(
        _node_body,
        out_shape=(
            jax.ShapeDtypeStruct((N, F), jnp.float32),
            jax.ShapeDtypeStruct((NP, F), jnp.float32),
        ),
    )(x, Wi, bi.reshape(1, F), Wj, bj.reshape(1, F))


def _sc_body(sl, g_hbm, hj_hbm, idxi_hbm, idxj_hbm, z_hbm, out_hbm,
             ii0, ii1, ii2, ii3, ij0, ij1, ij2, ij3,
             g0, g1, r0, r1, acc,
             isem0, isem1, isem2, isem3,
             lsem0, lsem1, gsem0, gsem1, ssem0, ssem1):
    c = lax.axis_index("c")
    s = lax.axis_index("s")
    wid = s * NC + c
    gbase = wid * EPW
    ebase = sl * ES + gbase
    nslice = pl.ds(s * RPT, RPT)
    iibufs = (ii0, ii1, ii2, ii3)
    ijbufs = (ij0, ij1, ij2, ij3)
    gbufs = (g0, g1)
    rbufs = (r0, r1)
    isems = (isem0, isem1, isem2, isem3)
    lsems = (lsem0, lsem1)
    gsems = (gsem0, gsem1)
    ssems = (ssem0, ssem1)

    pltpu.sync_copy(z_hbm.at[nslice], acc.at[nslice])
    plsc.subcore_barrier()

    def start_idx(k, q):
        off = pl.ds(ebase + k * B, B)
        pltpu.async_copy(idxj_hbm.at[off], ijbufs[q], isems[q])
        pltpu.async_copy(idxi_hbm.at[off], iibufs[q], isems[q])

    def wait_idx(q):
        pltpu.make_async_copy(idxj_hbm.at[pl.ds(0, B)], ijbufs[q],
                              isems[q]).wait()
        pltpu.make_async_copy(idxi_hbm.at[pl.ds(0, B)], iibufs[q],
                              isems[q]).wait()

    def start_inputs(k, d, q):
        pltpu.async_copy(hj_hbm.at[ijbufs[q]], rbufs[d], gsems[d])
        pltpu.async_copy(g_hbm.at[pl.ds(gbase + k * B, B)], gbufs[d], lsems[d])

    def wait_inputs(k, d, q):
        pltpu.make_async_copy(hj_hbm.at[ijbufs[q]], rbufs[d],
                              gsems[d]).wait()
        pltpu.make_async_copy(g_hbm.at[pl.ds(gbase + k * B, B)], gbufs[d],
                              lsems[d]).wait()

    def start_scatter(d, q):
        pltpu.async_copy(rbufs[d], acc.at[iibufs[q]], ssems[d], add=True)

    def wait_scatter(d, q):
        pltpu.make_async_copy(rbufs[d], acc.at[iibufs[q]],
                              ssems[d]).wait()

    start_idx(0, 0)
    start_idx(1, 1)
    wait_idx(0)
    start_inputs(0, 0, 0)

    def step(t, carry):
        kk = t * 4
        for b in range(4):
            k = kk + b
            d = b % 2

            @pl.when(k < CHUNKS)
            def _():
                wait_inputs(k, d, b)
                @pl.when(k + 2 < CHUNKS)
                def _():
                    start_idx(k + 2, (b + 2) % 4)

                @pl.when(k >= 1)
                def _():
                    wait_scatter(1 - d, (b + 3) % 4)

                @pl.when(k + 1 < CHUNKS)
                def _():
                    wait_idx((b + 1) % 4)
                    start_inputs(k + 1, 1 - d, (b + 1) % 4)

                @plsc.parallel_loop(0, B, 1, unroll=8)
                def _(i):
                    for cc in range(F // 16):
                        sli = pl.ds(cc * 16, 16)
                        rbufs[d][i, sli] = rbufs[d][i, sli] * gbufs[d][i, sli]

                start_scatter(d, b)

        return carry

    lax.fori_loop(0, (CHUNKS + 3) // 4, step, 0)
    wait_scatter((CHUNKS - 1) % 2, (CHUNKS - 1) % 4)
    plsc.subcore_barrier()
    pltpu.sync_copy(acc.at[nslice], out_hbm.at[c, nslice])


def _sc_gather_scatter(g, hj, idx_i, idx_j, zeros_nf, sl):
    mesh = plsc.VectorSubcoreMesh(core_axis_name="c", subcore_axis_name="s")
    f = pl.kernel(
        functools.partial(_sc_body, sl),
        out_type=jax.ShapeDtypeStruct((NC, NP, F), jnp.float32),
        mesh=mesh,
        scratch_types=(
            [pltpu.VMEM((B,), jnp.int32)] * 8
            + [pltpu.VMEM((B, F), jnp.float32)] * 4
            + [pltpu.VMEM_SHARED((NP, F), jnp.float32)]
            + [pltpu.SemaphoreType.DMA] * 10
        ),
    )
    return f(g, hj, idx_i, idx_j, zeros_nf)


def _fin_body(x_ref, xi_ref, p0_ref, w01, b01, w02, b02,
              w11, b11, w12, b12, wd, bd_, u_, out_ref):
    m = xi_ref[...] + p0_ref[0, :N, :] + p0_ref[1, :N, :]
    t = _dot_t(m, w01[...]) + b01[...]
    m = m + _dot_t(t, w02[...]) + b02[...]
    t = _dot_t(m, w11[...]) + b11[...]
    m = m + _dot_t(t, w12[...]) + b12[...]
    out_ref[...] = u_[...] * x_ref[...] + _dot_t(m, wd[...]) + bd_[...]


def _final(x, xi, p0, r0_W1, r0_b1, r0_W2, r0_b2,
           r1_W1, r1_b1, r1_W2, r1_b2, Wd, bd, u):
    return pl.pallas_call(
        _fin_body,
        out_shape=jax.ShapeDtypeStruct((N, F), jnp.float32),
    )(x, xi, p0, r0_W1, r0_b1.reshape(1, F), r0_W2, r0_b2.reshape(1, F),
      r1_W1, r1_b1.reshape(1, F), r1_W2, r1_b2.reshape(1, F),
      Wd, bd.reshape(1, F), u.reshape(1, F))


def kernel(x, rbf, idx_i, idx_j, Wk, Wi, bi, Wj, bj,
           r0_W1, r0_b1, r0_W2, r0_b2, r1_W1, r1_b1, r1_W2, r1_b2,
           Wd, bd, u):
    xi, hj = _node_matmuls(x, Wi, bi, Wj, bj)
    g0 = _edge_matmul(rbf, Wk, 0)
    zeros_nf = jnp.zeros((NP, F), dtype=jnp.float32)
    p0 = _sc_gather_scatter(g0, hj, idx_i, idx_j, zeros_nf, 0)
    return _final(x, xi, p0, r0_W1, r0_b1, r0_W2, r0_b2,
                  r1_W1, r1_b1, r1_W2, r1_b2, Wd, bd, u)


# speedup vs baseline: 1.0202x; 1.0202x over previous
import functools

import jax
import jax.numpy as jnp
from jax import lax
from jax.experimental import pallas as pl
from jax.experimental.pallas import tpu as pltpu
from jax.experimental.pallas import tpu_sc as plsc

N = 10000
E = 320000
F = 128
K = 64

NC = 2
NS = 16
NW = NC * NS
SLICE_E = (192000, 128000)
SLICE_BASE = (0, 192000)
B = 80
NP = 10240
RPT = NP // NS
BE = 4000


def _dot_t(a, w):
    return lax.dot_general(a, w, (((1,), (1,)), ((), ())),
                           preferred_element_type=jnp.float32)


def _g_body(rbf_ref, wk_ref, out_ref):
    out_ref[...] = _dot_t(rbf_ref[...], wk_ref[...])


def _edge_matmul(rbf, Wk, sl):
    base = SLICE_BASE[sl] // BE
    es = SLICE_E[sl]
    return pl.pallas_call(
        _g_body,
        grid=(es // BE,),
        in_specs=[
            pl.BlockSpec((BE, K), lambda i: (base + i, 0)),
            pl.BlockSpec((F, K), lambda i: (0, 0)),
        ],
        out_specs=pl.BlockSpec((BE, F), lambda i: (i, 0)),
        out_shape=jax.ShapeDtypeStruct((es, F), jnp.float32),
    )(rbf, Wk)


def _node_body(x_ref, wi_ref, bi_ref, wj_ref, bj_ref, xi_ref, hj_ref):
    xv = x_ref[...]
    xi_ref[...] = _dot_t(xv, wi_ref[...]) + bi_ref[...]
    hj_ref[:N, :] = _dot_t(xv, wj_ref[...]) + bj_ref[...]
    hj_ref[N:, :] = jnp.zeros((NP - N, F), jnp.float32)


def _node_matmuls(x, Wi, bi, Wj, bj):
    return pl.pallas_call(
        _node_body,
        out_shape=(
            jax.ShapeDtypeStruct((N, F), jnp.float32),
            jax.ShapeDtypeStruct((NP, F), jnp.float32),
        ),
    )(x, Wi, bi.reshape(1, F), Wj, bj.reshape(1, F))


def _sc_body(sl, g_hbm, hj_hbm, idxi_hbm, idxj_hbm, z_hbm, out_hbm,
             ii0, ii1, ii2, ii3, ij0, ij1, ij2, ij3,
             g0, g1, r0, r1, acc,
             isem0, isem1, isem2, isem3,
             lsem0, lsem1, gsem0, gsem1, ssem0, ssem1):
    epw = SLICE_E[sl] // NW
    CHUNKS = epw // B
    c = lax.axis_index("c")
    s = lax.axis_index("s")
    wid = s * NC + c
    gbase = wid * epw
    ebase = SLICE_BASE[sl] + gbase
    nslice = pl.ds(s * RPT, RPT)
    iibufs = (ii0, ii1, ii2, ii3)
    ijbufs = (ij0, ij1, ij2, ij3)
    gbufs = (g0, g1)
    rbufs = (r0, r1)
    isems = (isem0, isem1, isem2, isem3)
    lsems = (lsem0, lsem1)
    gsems = (gsem0, gsem1)
    ssems = (ssem0, ssem1)

    pltpu.sync_copy(z_hbm.at[nslice], acc.at[nslice])
    plsc.subcore_barrier()

    def start_idx(k, q):
        off = pl.ds(ebase + k * B, B)
        pltpu.async_copy(idxj_hbm.at[off], ijbufs[q], isems[q])
        pltpu.async_copy(idxi_hbm.at[off], iibufs[q], isems[q])

    def wait_idx(q):
        pltpu.make_async_copy(idxj_hbm.at[pl.ds(0, B)], ijbufs[q],
                              isems[q]).wait()
        pltpu.make_async_copy(idxi_hbm.at[pl.ds(0, B)], iibufs[q],
                              isems[q]).wait()

    def start_inputs(k, d, q):
        pltpu.async_copy(hj_hbm.at[ijbufs[q]], rbufs[d], gsems[d])
        pltpu.async_copy(g_hbm.at[pl.ds(gbase + k * B, B)], gbufs[d], lsems[d])

    def wait_inputs(k, d, q):
        pltpu.make_async_copy(hj_hbm.at[ijbufs[q]], rbufs[d],
                              gsems[d]).wait()
        pltpu.make_async_copy(g_hbm.at[pl.ds(gbase + k * B, B)], gbufs[d],
                              lsems[d]).wait()

    def start_scatter(d, q):
        pltpu.async_copy(rbufs[d], acc.at[iibufs[q]], ssems[d], add=True)

    def wait_scatter(d, q):
        pltpu.make_async_copy(rbufs[d], acc.at[iibufs[q]],
                              ssems[d]).wait()

    start_idx(0, 0)
    start_idx(1, 1)
    wait_idx(0)
    start_inputs(0, 0, 0)

    def step(t, carry):
        kk = t * 4
        for b in range(4):
            k = kk + b
            d = b % 2

            @pl.when(k < CHUNKS)
            def _():
                wait_inputs(k, d, b)
                @pl.when(k + 2 < CHUNKS)
                def _():
                    start_idx(k + 2, (b + 2) % 4)

                @pl.when(k >= 1)
                def _():
                    wait_scatter(1 - d, (b + 3) % 4)

                @pl.when(k + 1 < CHUNKS)
                def _():
                    wait_idx((b + 1) % 4)
                    start_inputs(k + 1, 1 - d, (b + 1) % 4)

                @plsc.parallel_loop(0, B, 1, unroll=4)
                def _(i):
                    for cc in range(F // 16):
                        sli = pl.ds(cc * 16, 16)
                        rbufs[d][i, sli] = rbufs[d][i, sli] * gbufs[d][i, sli]

                start_scatter(d, b)

        return carry

    lax.fori_loop(0, (CHUNKS + 3) // 4, step, 0)
    wait_scatter((CHUNKS - 1) % 2, (CHUNKS - 1) % 4)
    plsc.subcore_barrier()
    pltpu.sync_copy(acc.at[nslice], out_hbm.at[c, nslice])


def _sc_gather_scatter(g, hj, idx_i, idx_j, zeros_nf, sl):
    mesh = plsc.VectorSubcoreMesh(core_axis_name="c", subcore_axis_name="s")
    f = pl.kernel(
        functools.partial(_sc_body, sl),
        out_type=jax.ShapeDtypeStruct((NC, NP, F), jnp.float32),
        mesh=mesh,
        scratch_types=(
            [pltpu.VMEM((B,), jnp.int32)] * 8
            + [pltpu.VMEM((B, F), jnp.float32)] * 4
            + [pltpu.VMEM_SHARED((NP, F), jnp.float32)]
            + [pltpu.SemaphoreType.DMA] * 10
        ),
    )
    return f(g, hj, idx_i, idx_j, zeros_nf)


def _fin_body(x_ref, xi_ref, p0_ref, p1_ref, w01, b01, w02, b02,
              w11, b11, w12, b12, wd, bd_, u_, out_ref):
    m = (xi_ref[...] + p0_ref[0, :N, :] + p0_ref[1, :N, :]
         + p1_ref[0, :N, :] + p1_ref[1, :N, :])
    t = _dot_t(m, w01[...]) + b01[...]
    m = m + _dot_t(t, w02[...]) + b02[...]
    t = _dot_t(m, w11[...]) + b11[...]
    m = m + _dot_t(t, w12[...]) + b12[...]
    out_ref[...] = u_[...] * x_ref[...] + _dot_t(m, wd[...]) + bd_[...]


def _final(x, xi, p0, p1, r0_W1, r0_b1, r0_W2, r0_b2,
           r1_W1, r1_b1, r1_W2, r1_b2, Wd, bd, u):
    return pl.pallas_call(
        _fin_body,
        out_shape=jax.ShapeDtypeStruct((N, F), jnp.float32),
    )(x, xi, p0, p1, r0_W1, r0_b1.reshape(1, F), r0_W2, r0_b2.reshape(1, F),
      r1_W1, r1_b1.reshape(1, F), r1_W2, r1_b2.reshape(1, F),
      Wd, bd.reshape(1, F), u.reshape(1, F))


def kernel(x, rbf, idx_i, idx_j, Wk, Wi, bi, Wj, bj,
           r0_W1, r0_b1, r0_W2, r0_b2, r1_W1, r1_b1, r1_W2, r1_b2,
           Wd, bd, u):
    xi, hj = _node_matmuls(x, Wi, bi, Wj, bj)
    g0 = _edge_matmul(rbf, Wk, 0)
    zeros_nf = jnp.zeros((NP, F), dtype=jnp.float32)
    p0 = _sc_gather_scatter(g0, hj, idx_i, idx_j, zeros_nf, 0)
    g1 = _edge_matmul(rbf, Wk, 1)
    p1 = _sc_gather_scatter(g1, hj, idx_i, idx_j, zeros_nf, 1)
    return _final(x, xi, p0, p1, r0_W1, r0_b1, r0_W2, r0_b2,
                  r1_W1, r1_b1, r1_W2, r1_b2, Wd, bd, u)
